# baseline (speedup 1.0000x reference)
import jax
import jax.numpy as jnp
from jax.experimental import pallas as pl

_R = (0, 46, 167, 100, 191, 220, 0, 10)
_G = (160, 141, 0, 62, 30, 87, 166, 91)
_B = (177, 239, 174, 191, 75, 46, 0, 196)
_PACKED_LUT = tuple(r | (g << 8) | (b << 16) for r, g, b in zip(_R, _G, _B))

_HBLK = 32


def _lsh_kernel(rot_ref, x_ref, r_ref, g_ref, b_ref):
    for row in range(x_ref.shape[2]):
        v = jax.lax.dot_general(
            rot_ref[...], x_ref[0, :, row, :],
            dimension_numbers=(((1,), (0,)), ((), ())),
            preferred_element_type=jnp.float32,
        )
        best = v[0:16, :]
        packed = jnp.full(best.shape, _PACKED_LUT[0], dtype=jnp.int32)
        for k in range(1, 8):
            s = v[16 * k:16 * (k + 1), :]
            gt = s > best
            packed = jnp.where(gt, _PACKED_LUT[k], packed)
            best = jnp.maximum(best, s)
        r_ref[0, row] = (packed & 0xFF).astype(jnp.uint8).T
        g_ref[0, row] = ((packed >> 8) & 0xFF).astype(jnp.uint8).T
        b_ref[0, row] = ((packed >> 16) & 0xFF).astype(jnp.uint8).T


def kernel(inp, rotations):
    B, C, H, W = inp.shape
    n_hashes = rotations.shape[1]
    rot = jnp.transpose(rotations, (0, 2, 1)).reshape(C, 4 * n_hashes)
    rot_packed = jnp.concatenate([rot, -rot], axis=1).T

    out_sds = jax.ShapeDtypeStruct((B, H, W, n_hashes), jnp.uint8)
    return pl.pallas_call(
        _lsh_kernel,
        grid=(H // _HBLK,),
        in_specs=[
            pl.BlockSpec((128, C), lambda i: (0, 0)),
            pl.BlockSpec((1, C, _HBLK, W), lambda i: (0, 0, i, 0)),
        ],
        out_specs=[pl.BlockSpec((1, _HBLK, W, n_hashes), lambda i: (0, i, 0, 0))] * 3,
        out_shape=[out_sds, out_sds, out_sds],
    )(rot_packed, inp)


# speedup vs baseline: 1.0135x; 1.0135x over previous
import jax
import jax.numpy as jnp
from jax.experimental import pallas as pl

_R = (0, 46, 167, 100, 191, 220, 0, 10)
_G = (160, 141, 0, 62, 30, 87, 166, 91)
_B = (177, 239, 174, 191, 75, 46, 0, 196)
_PACKED_LUT = tuple(r | (g << 8) | (b << 16) for r, g, b in zip(_R, _G, _B))

_HBLK = 16


def _lsh_kernel(rot_ref, x_ref, r_ref, g_ref, b_ref):
    for row in range(x_ref.shape[2]):
        v = jax.lax.dot_general(
            rot_ref[...], x_ref[0, :, row, :],
            dimension_numbers=(((1,), (0,)), ((), ())),
            preferred_element_type=jnp.float32,
        )
        best = v[0:16, :]
        packed = jnp.full(best.shape, _PACKED_LUT[0], dtype=jnp.int32)
        for k in range(1, 8):
            s = v[16 * k:16 * (k + 1), :]
            gt = s > best
            packed = jnp.where(gt, _PACKED_LUT[k], packed)
            best = jnp.maximum(best, s)
        r_ref[0, row] = (packed & 0xFF).astype(jnp.uint8).T
        g_ref[0, row] = ((packed >> 8) & 0xFF).astype(jnp.uint8).T
        b_ref[0, row] = ((packed >> 16) & 0xFF).astype(jnp.uint8).T


def kernel(inp, rotations):
    B, C, H, W = inp.shape
    n_hashes = rotations.shape[1]
    rot = jnp.transpose(rotations, (0, 2, 1)).reshape(C, 4 * n_hashes)
    rot_packed = jnp.concatenate([rot, -rot], axis=1).T

    out_sds = jax.ShapeDtypeStruct((B, H, W, n_hashes), jnp.uint8)
    return pl.pallas_call(
        _lsh_kernel,
        grid=(H // _HBLK,),
        in_specs=[
            pl.BlockSpec((128, C), lambda i: (0, 0)),
            pl.BlockSpec((1, C, _HBLK, W), lambda i: (0, 0, i, 0)),
        ],
        out_specs=[pl.BlockSpec((1, _HBLK, W, n_hashes), lambda i: (0, i, 0, 0))] * 3,
        out_shape=[out_sds, out_sds, out_sds],
    )(rot_packed, inp)


# speedup vs baseline: 1.9414x; 1.9154x over previous
import jax
import jax.numpy as jnp
from jax.experimental import pallas as pl

_R = (0, 46, 167, 100, 191, 220, 0, 10)
_G = (160, 141, 0, 62, 30, 87, 166, 91)
_B = (177, 239, 174, 191, 75, 46, 0, 196)
_PACKED_LUT = tuple(r | (g << 8) | (b << 16) for r, g, b in zip(_R, _G, _B))

_HBLK = 16


def _lsh_kernel(rot_ref, x_ref, r_ref, g_ref, b_ref):
    for row in range(x_ref.shape[2]):
        v = jax.lax.dot_general(
            rot_ref[...], x_ref[0, :, row, :],
            dimension_numbers=(((1,), (0,)), ((), ())),
            preferred_element_type=jnp.float32,
        )
        best = v[0:16, :]
        packed = jnp.full(best.shape, _PACKED_LUT[0], dtype=jnp.int32)
        for k in range(1, 8):
            s = v[16 * k:16 * (k + 1), :]
            gt = s > best
            packed = jnp.where(gt, _PACKED_LUT[k], packed)
            best = jnp.maximum(best, s)
        r_ref[0, row] = (packed & 0xFF).astype(jnp.uint8)
        g_ref[0, row] = ((packed >> 8) & 0xFF).astype(jnp.uint8)
        b_ref[0, row] = ((packed >> 16) & 0xFF).astype(jnp.uint8)


def kernel(inp, rotations):
    B, C, H, W = inp.shape
    n_hashes = rotations.shape[1]
    rot = jnp.transpose(rotations, (0, 2, 1)).reshape(C, 4 * n_hashes)
    rot_packed = jnp.concatenate([rot, -rot], axis=1).T

    out_sds = jax.ShapeDtypeStruct((B, H, n_hashes, W), jnp.uint8)
    r, g, b = pl.pallas_call(
        _lsh_kernel,
        grid=(H // _HBLK,),
        in_specs=[
            pl.BlockSpec((128, C), lambda i: (0, 0)),
            pl.BlockSpec((1, C, _HBLK, W), lambda i: (0, 0, i, 0)),
        ],
        out_specs=[pl.BlockSpec((1, _HBLK, n_hashes, W), lambda i: (0, i, 0, 0))] * 3,
        out_shape=[out_sds, out_sds, out_sds],
    )(rot_packed, inp)
    perm = (0, 1, 3, 2)
    return (
        jnp.transpose(r, perm),
        jnp.transpose(g, perm),
        jnp.transpose(b, perm),
    )


# speedup vs baseline: 1.9439x; 1.0013x over previous
import jax
import jax.numpy as jnp
from jax.experimental import pallas as pl
from jax.experimental.pallas import tpu as pltpu

_R = (0, 46, 167, 100, 191, 220, 0, 10)
_G = (160, 141, 0, 62, 30, 87, 166, 91)
_B = (177, 239, 174, 191, 75, 46, 0, 196)
_PACKED_LUT = tuple(r | (g << 8) | (b << 16) for r, g, b in zip(_R, _G, _B))

_HBLK = 16


def _lsh_kernel(rot_ref, x_ref, r_ref, g_ref, b_ref):
    for row in range(x_ref.shape[2]):
        v = jax.lax.dot_general(
            rot_ref[...], x_ref[0, :, row, :],
            dimension_numbers=(((1,), (0,)), ((), ())),
            preferred_element_type=jnp.float32,
        )
        best = v[0:16, :]
        packed = jnp.full(best.shape, _PACKED_LUT[0], dtype=jnp.int32)
        for k in range(1, 8):
            s = v[16 * k:16 * (k + 1), :]
            gt = s > best
            packed = jnp.where(gt, _PACKED_LUT[k], packed)
            best = jnp.maximum(best, s)
        r_ref[0, row] = (packed & 0xFF).astype(jnp.uint8)
        g_ref[0, row] = ((packed >> 8) & 0xFF).astype(jnp.uint8)
        b_ref[0, row] = ((packed >> 16) & 0xFF).astype(jnp.uint8)


def kernel(inp, rotations):
    B, C, H, W = inp.shape
    n_hashes = rotations.shape[1]
    rot = jnp.transpose(rotations, (0, 2, 1)).reshape(C, 4 * n_hashes)
    rot_packed = jnp.concatenate([rot, -rot], axis=1).T

    out_sds = jax.ShapeDtypeStruct((B, H, n_hashes, W), jnp.uint8)
    r, g, b = pl.pallas_call(
        _lsh_kernel,
        grid=(H // _HBLK,),
        in_specs=[
            pl.BlockSpec((128, C), lambda i: (0, 0)),
            pl.BlockSpec((1, C, _HBLK, W), lambda i: (0, 0, i, 0)),
        ],
        out_specs=[pl.BlockSpec((1, _HBLK, n_hashes, W), lambda i: (0, i, 0, 0))] * 3,
        out_shape=[out_sds, out_sds, out_sds],
        compiler_params=pltpu.CompilerParams(dimension_semantics=("parallel",)),
    )(rot_packed, inp)
    perm = (0, 1, 3, 2)
    return (
        jnp.transpose(r, perm),
        jnp.transpose(g, perm),
        jnp.transpose(b, perm),
    )
